# baseline (speedup 1.0000x reference)
import functools
import jax
import jax.numpy as jnp
from jax import lax
from jax.experimental import pallas as pl
from jax.experimental.pallas import tpu as pltpu
from jax.experimental.pallas import tpu_sc as plsc

B = 8
N = 2048
NC, NS, L = 2, 16, 16
NW = NC * NS
Q = NW // B
COLS_W = N // Q
CB = 8
BLOCKS = COLS_W // (CB * L)

_mesh = plsc.VectorSubcoreMesh(core_axis_name="c", subcore_axis_name="s")


def _min_pass(src_planes, dst_planes, out_v, q):
    sx_v, sy_v, sz_v, sn_v = src_planes
    dx_v, dy_v, dz_v, dn_v = dst_planes
    for blk in range(BLOCKS):
        colbase = q * COLS_W + blk * (CB * L)
        gx = [dx_v[pl.ds(colbase + v * L, L)] * -2.0 for v in range(CB)]
        gy = [dy_v[pl.ds(colbase + v * L, L)] * -2.0 for v in range(CB)]
        gz = [dz_v[pl.ds(colbase + v * L, L)] * -2.0 for v in range(CB)]
        init = tuple(jnp.full((L,), jnp.inf, jnp.float32) for _ in range(CB))

        def body(i, acc):
            ridx = jnp.full((L,), i, jnp.int32)
            px = plsc.load_gather(sx_v, [ridx])
            py = plsc.load_gather(sy_v, [ridx])
            pz = plsc.load_gather(sz_v, [ridx])
            pn = plsc.load_gather(sn_v, [ridx])
            return tuple(
                jnp.minimum(acc[v],
                            px * gx[v] + py * gy[v] + pz * gz[v] + pn)
                for v in range(CB))

        acc = lax.fori_loop(0, N, body, init)
        for v in range(CB):
            gn = dn_v[pl.ds(colbase + v * L, L)]
            out_v[pl.ds(blk * CB * L + v * L, L)] = acc[v] + gn


@functools.partial(
    pl.kernel,
    out_type=[jax.ShapeDtypeStruct((B, N), jnp.float32),
              jax.ShapeDtypeStruct((B, N), jnp.float32)],
    mesh=_mesh,
    scratch_types=(
        [pltpu.VMEM((N,), jnp.float32) for _ in range(8)]
        + [pltpu.VMEM((COLS_W,), jnp.float32),
           pltpu.VMEM((COLS_W,), jnp.float32)]
    ),
    compiler_params=pltpu.CompilerParams(needs_layout_passes=False),
)
def _sc_minsq(p_hbm, g_hbm, z1_hbm, z2_hbm,
              px_v, py_v, pz_v, pn_v, gx_v, gy_v, gz_v, gn_v, o1_v, o2_v):
    wid = lax.axis_index("c") * NS + lax.axis_index("s")
    b = wid // Q
    q = wid % Q
    p_planes = (px_v, py_v, pz_v, pn_v)
    g_planes = (gx_v, gy_v, gz_v, gn_v)
    for c in range(4):
        pltpu.sync_copy(p_hbm.at[b, c], p_planes[c])
        pltpu.sync_copy(g_hbm.at[b, c], g_planes[c])
    _min_pass(p_planes, g_planes, o1_v, q)
    pltpu.sync_copy(o1_v, z1_hbm.at[b, pl.ds(q * COLS_W, COLS_W)])
    _min_pass(g_planes, p_planes, o2_v, q)
    pltpu.sync_copy(o2_v, z2_hbm.at[b, pl.ds(q * COLS_W, COLS_W)])


def _finish_body(z1_ref, z2_ref, out_ref):
    s = (jnp.sum(jnp.sqrt(jnp.maximum(z1_ref[...], 0.0)))
         + jnp.sum(jnp.sqrt(jnp.maximum(z2_ref[...], 0.0))))
    out_ref[0, 0] = s * (1.0 / (B * N))


def _finish(z1, z2):
    out = pl.pallas_call(
        _finish_body,
        out_specs=pl.BlockSpec(memory_space=pltpu.SMEM),
        out_shape=jax.ShapeDtypeStruct((1, 1), jnp.float32),
    )(z1, z2)
    return out[0, 0]


def kernel(predict_pc, gt_pc):
    pp = jnp.concatenate(
        [predict_pc, jnp.sum(predict_pc * predict_pc, axis=1, keepdims=True)],
        axis=1)
    gp = jnp.concatenate(
        [gt_pc, jnp.sum(gt_pc * gt_pc, axis=1, keepdims=True)], axis=1)
    z1, z2 = _sc_minsq(pp, gp)
    return _finish(z1, z2)
